# baseline (speedup 1.0000x reference)
import functools

import jax
import jax.numpy as jnp
from jax import lax
from jax.experimental import pallas as pl
from jax.experimental.pallas import tpu as pltpu
from jax.experimental.pallas import tpu_sc as plsc

VOCAB = 100000
DIM = 64
B = 4096
L = 50

TOTAL = B * L
NC = 2
NS = 16
NW = NC * NS
PER_W = TOTAL // NW
CHUNK = 128
NCH = PER_W // CHUNK
K = 5
NG = NCH // K

TBL_BLK = 2048
TBL_GRID = -(-VOCAB // TBL_BLK)


def _transform_body(tt_ref, w_ref, b_ref, out_ref):
    x = lax.dot_general(
        tt_ref[...], w_ref[...],
        dimension_numbers=(((0,), (1,)), ((), ())),
        preferred_element_type=jnp.float32,
    )
    y = jax.nn.sigmoid(x + b_ref[...])
    out_ref[...] = jnp.concatenate([y, y], axis=1)


def _transform_table(tt, W, b2d):
    return pl.pallas_call(
        _transform_body,
        grid=(TBL_GRID,),
        in_specs=[
            pl.BlockSpec((DIM, TBL_BLK), lambda i: (0, i)),
            pl.BlockSpec((DIM, DIM), lambda i: (0, 0)),
            pl.BlockSpec((1, DIM), lambda i: (0, 0)),
        ],
        out_specs=pl.BlockSpec((TBL_BLK, 2 * DIM), lambda i: (i, 0)),
        out_shape=jax.ShapeDtypeStruct((TBL_GRID * TBL_BLK, 2 * DIM), jnp.float32),
    )(tt, W, b2d)


_sc_mesh = plsc.VectorSubcoreMesh(core_axis_name="c", subcore_axis_name="s")


@functools.partial(
    pl.kernel,
    out_type=jax.ShapeDtypeStruct((TOTAL, DIM), jnp.float32),
    mesh=_sc_mesh,
    scratch_types=[
        pltpu.VMEM((NCH, CHUNK), jnp.int32),
        pltpu.VMEM((2, K * CHUNK, DIM), jnp.float32),
        pltpu.SemaphoreType.DMA,
        pltpu.SemaphoreType.DMA,
    ],
    compiler_params=pltpu.CompilerParams(use_tc_tiling_on_sc=False),
)
def _sc_gather(tprime_hbm, idx_hbm, out_hbm, idx_v, rows_v, sem0, sem1):
    wid = lax.axis_index("s") * NC + lax.axis_index("c")
    base = wid * PER_W
    pltpu.sync_copy(idx_hbm.at[wid], idx_v)

    def fire(g, buf, sem):
        return [
            pltpu.async_copy(
                tprime_hbm.at[idx_v.at[g * K + j]],
                rows_v.at[buf].at[pl.ds(j * CHUNK, CHUNK)], sem)
            for j in range(K)
        ]

    @pl.loop(0, NG, step=2)
    def _groups(e):
        h0 = fire(e, 0, sem0)
        h1 = fire(e + 1, 1, sem1)
        for h in h0:
            h.wait()
        pltpu.sync_copy(rows_v.at[0],
                        out_hbm.at[pl.ds(base + e * K * CHUNK, K * CHUNK)])
        for h in h1:
            h.wait()
        pltpu.sync_copy(rows_v.at[1],
                        out_hbm.at[pl.ds(base + (e + 1) * K * CHUNK, K * CHUNK)])


NB = 4
NBLK = B // NB


def _transpose_body(g_ref, out_ref):
    x = g_ref[...]
    xt = jnp.transpose(x)
    out_ref[...] = jnp.concatenate([xt[0:DIM, :], xt[DIM:2 * DIM, :]],
                                   axis=1)[None]


def _transpose_out(g128):
    return pl.pallas_call(
        _transpose_body,
        grid=(L, NB),
        in_specs=[
            pl.BlockSpec((NBLK // 2, 2 * DIM), lambda l, nb: (l * NB + nb, 0)),
        ],
        out_specs=pl.BlockSpec((1, DIM, NBLK), lambda l, nb: (l, 0, nb)),
        out_shape=jax.ShapeDtypeStruct((L, DIM, B), jnp.float32),
    )(g128)


def kernel(batch, table, W, b):
    tt = jnp.transpose(table)
    t128 = _transform_table(tt, W, b.reshape(1, DIM))
    tlin = t128.reshape(TBL_GRID * TBL_BLK * 2, DIM)
    bt = jnp.transpose(batch).astype(jnp.int32) * 2
    bperm = bt.reshape(L, NB, 2, NBLK // 2).transpose(0, 1, 3, 2)
    idx = bperm.reshape(NW, NCH, CHUNK)
    gathered = _sc_gather(tlin, idx)
    out3 = _transpose_out(gathered.reshape(TOTAL // 2, 2 * DIM))
    return jnp.transpose(out3, (2, 0, 1))


# speedup vs baseline: 1.3279x; 1.3279x over previous
import functools

import jax
import jax.numpy as jnp
from jax import lax
from jax.experimental import pallas as pl
from jax.experimental.pallas import tpu as pltpu
from jax.experimental.pallas import tpu_sc as plsc

VOCAB = 100000
DIM = 64
B = 4096
L = 50

TOTAL = B * L
NC = 2
NS = 16
NW = NC * NS
PER_W = TOTAL // NW
CHUNK = 128
NCH = PER_W // CHUNK
K = 5
NG = NCH // K

TBL_BLK = 2048
TBL_GRID = -(-VOCAB // TBL_BLK)


def _transform_body(tt_ref, w_ref, b_ref, out_ref):
    x = lax.dot_general(
        tt_ref[...], w_ref[...],
        dimension_numbers=(((0,), (1,)), ((), ())),
        preferred_element_type=jnp.float32,
    )
    y = jax.nn.sigmoid(x + b_ref[...])
    out_ref[...] = jnp.concatenate([y, y], axis=1)


def _transform_table(tt, W, b2d):
    return pl.pallas_call(
        _transform_body,
        grid=(TBL_GRID,),
        in_specs=[
            pl.BlockSpec((DIM, TBL_BLK), lambda i: (0, i)),
            pl.BlockSpec((DIM, DIM), lambda i: (0, 0)),
            pl.BlockSpec((1, DIM), lambda i: (0, 0)),
        ],
        out_specs=pl.BlockSpec((TBL_BLK, 2 * DIM), lambda i: (i, 0)),
        out_shape=jax.ShapeDtypeStruct((TBL_GRID * TBL_BLK, 2 * DIM), jnp.float32),
    )(tt, W, b2d)


_sc_mesh = plsc.VectorSubcoreMesh(core_axis_name="c", subcore_axis_name="s")


@functools.partial(
    pl.kernel,
    out_type=jax.ShapeDtypeStruct((TOTAL, DIM), jnp.float32),
    mesh=_sc_mesh,
    scratch_types=[
        pltpu.VMEM((NCH, CHUNK), jnp.int32),
        pltpu.VMEM((2, K * CHUNK, DIM), jnp.float32),
        pltpu.SemaphoreType.DMA,
        pltpu.SemaphoreType.DMA,
    ],
    compiler_params=pltpu.CompilerParams(use_tc_tiling_on_sc=False),
)
def _sc_gather(tprime_hbm, idx_hbm, out_hbm, idx_v, rows_v, sem0, sem1):
    wid = lax.axis_index("s") * NC + lax.axis_index("c")
    base = wid * PER_W
    pltpu.sync_copy(idx_hbm.at[wid], idx_v)

    def fire(g, buf, sem):
        return [
            pltpu.async_copy(
                tprime_hbm.at[idx_v.at[g * K + j]],
                rows_v.at[buf].at[pl.ds(j * CHUNK, CHUNK)], sem)
            for j in range(K)
        ]

    @pl.loop(0, NG, step=2)
    def _groups(e):
        h0 = fire(e, 0, sem0)
        h1 = fire(e + 1, 1, sem1)
        for h in h0:
            h.wait()
        pltpu.sync_copy(rows_v.at[0],
                        out_hbm.at[pl.ds(base + e * K * CHUNK, K * CHUNK)])
        for h in h1:
            h.wait()
        pltpu.sync_copy(rows_v.at[1],
                        out_hbm.at[pl.ds(base + (e + 1) * K * CHUNK, K * CHUNK)])


NB = 4
NBLK = B // NB


def _transpose_body(g_ref, out_ref):
    x = g_ref[...]
    xt = jnp.transpose(x)
    out_ref[...] = jnp.concatenate([xt[0:DIM, :], xt[DIM:2 * DIM, :]],
                                   axis=1)[None]


def _transpose_out(g128):
    return pl.pallas_call(
        _transpose_body,
        grid=(L,),
        in_specs=[
            pl.BlockSpec((B // 2, 2 * DIM), lambda l: (l, 0)),
        ],
        out_specs=pl.BlockSpec((1, DIM, B), lambda l: (l, 0, 0)),
        out_shape=jax.ShapeDtypeStruct((L, DIM, B), jnp.float32),
    )(g128)


def kernel(batch, table, W, b):
    tt = jnp.transpose(table)
    t128 = _transform_table(tt, W, b.reshape(1, DIM))
    tlin = t128.reshape(TBL_GRID * TBL_BLK * 2, DIM)
    bt = jnp.transpose(batch).astype(jnp.int32) * 2
    bperm = bt.reshape(L, 2, B // 2).transpose(0, 2, 1)
    idx = bperm.reshape(NW, NCH, CHUNK)
    gathered = _sc_gather(tlin, idx)
    out3 = _transpose_out(gathered.reshape(TOTAL // 2, 2 * DIM))
    return jnp.transpose(out3, (2, 0, 1))
